# baseline (speedup 1.0000x reference)
import functools

import jax
import jax.numpy as jnp
from jax import lax
from jax.experimental import pallas as pl
from jax.experimental.pallas import tpu as pltpu
from jax.experimental.pallas import tpu_sc as plsc

N_NODES = 10000
D = 128
N_EDGES = 320000

NC = 2
NS = 16
NW = NC * NS
EDGES_A = 9984
CHUNK = 64
SEGMENTS = ((0, 52), (52, 52), (104, 52))
SEG_EXTRA = (156, 8)
SEG_MAX = 52
ROWS_A = 632
ROWS_LAST = N_NODES - (NS - 1) * ROWS_A
ZROWS = 8
HR = 80


def _sc_accumulate(x, ei):
    mesh = plsc.VectorSubcoreMesh(core_axis_name="c", subcore_axis_name="s")

    @functools.partial(
        pl.kernel,
        out_type=(
            jax.ShapeDtypeStruct((NC, N_NODES, D), jnp.float32),
            jax.ShapeDtypeStruct((NW, HR, D), jnp.float32),
        ),
        mesh=mesh,
        compiler_params=pltpu.CompilerParams(needs_layout_passes=False),
        scratch_types=[
            pltpu.VMEM_SHARED((N_NODES, D), jnp.float32),
            pltpu.VMEM((2, SEG_MAX * CHUNK), jnp.int32),
            pltpu.VMEM((CHUNK, D), jnp.float32),
            pltpu.VMEM((CHUNK, D), jnp.float32),
            pltpu.VMEM((CHUNK, D), jnp.float32),
            pltpu.VMEM((CHUNK, D), jnp.float32),
            pltpu.VMEM((HR, D), jnp.float32),
            pltpu.SemaphoreType.DMA,
            pltpu.SemaphoreType.DMA,
            pltpu.SemaphoreType.DMA,
            pltpu.SemaphoreType.DMA,
            pltpu.SemaphoreType.DMA,
            pltpu.SemaphoreType.DMA,
            pltpu.SemaphoreType.DMA,
            pltpu.SemaphoreType.DMA,
        ],
    )
    def sc_kernel(x_hbm, ei_hbm,
                  feat_out, hist_out,
                  feat_acc, idx2,
                  rows_v0, rows_v1, rows_v2, rows_v3, hist_v,
                  sem0, sem1, sem2, sem3, ssem0, ssem1, ssem2, ssem3):
        cid = lax.axis_index("c")
        sid = lax.axis_index("s")
        wid = cid * NS + sid

        zvec = jnp.zeros((16,), jnp.float32)
        for i in range(ZROWS):
            for j in range(D // 16):
                rows_v0[i, pl.ds(j * 16, 16)] = zvec

        def zero_hist(i, _):
            for j in range(D // 16):
                hist_v[i, pl.ds(j * 16, 16)] = zvec
            return 0
        lax.fori_loop(0, HR, zero_hist, 0)

        row0 = sid * ROWS_A

        def zero_rows(nrows):
            def zbody(k, _):
                pltpu.sync_copy(rows_v0.at[pl.ds(0, ZROWS)],
                                feat_acc.at[pl.ds(row0 + k * ZROWS, ZROWS)])
                return 0
            lax.fori_loop(0, nrows // ZROWS, zbody, 0)

        pl.when(sid < NS - 1)(lambda: zero_rows(ROWS_A))
        pl.when(sid == NS - 1)(lambda: zero_rows(ROWS_LAST))
        plsc.subcore_barrier()

        edge_base = wid * EDGES_A
        ones16 = jnp.ones((16,), jnp.float32)
        bufs = ((rows_v0, sem0, ssem0), (rows_v1, sem1, ssem1),
                (rows_v2, sem2, ssem2), (rows_v3, sem3, ssem3))
        NB = len(bufs)

        def start_gather(off, b):
            rows_v, sem, _ = bufs[b]
            pltpu.async_copy(
                x_hbm.at[idx2.at[0].at[pl.ds(off * CHUNK, CHUNK)]],
                rows_v, sem)

        def drain_scatter(b):
            rows_v, _, ssem = bufs[b]
            for k in range(CHUNK // 16):
                pltpu.make_async_copy(x_hbm.at[pl.ds(0, 16)],
                                      rows_v.at[pl.ds(k * 16, 16)], ssem).wait()

        def finish_chunk(off, b, nxt_off, start_next, drain_prev):
            rows_v, sem, ssem = bufs[b]
            pltpu.make_async_copy(x_hbm.at[pl.ds(0, CHUNK)], rows_v, sem).wait()
            for k in range(CHUNK // 16):
                dv = idx2[1, pl.ds(off * CHUNK + k * 16, 16)]
                pltpu.async_copy(rows_v.at[pl.ds(k * 16, 16)],
                                 feat_acc.at[dv], ssem, add=True)
                hi = lax.shift_right_logical(dv, 7)
                lo = lax.bitwise_and(dv, 127)
                plsc.addupdate_scatter(hist_v, [hi, lo], ones16)
            if drain_prev:
                drain_scatter((b + 2) % NB)
            if start_next:
                start_gather(nxt_off, (b + 2) % NB)

        def run_segment(base, n):
            pltpu.sync_copy(
                ei_hbm.at[:, pl.ds(edge_base + base * CHUNK, n * CHUNK)],
                idx2.at[:, pl.ds(0, n * CHUNK)])
            start_gather(0, 0)
            start_gather(1, 1)
            finish_chunk(0, 0, 2, n > 2, False)
            finish_chunk(1, 1, 3, n > 3, False)
            m4 = (n - 4) // NB if n > 4 else 0
            rem = (n - 2) - NB * m4

            def body(j, _):
                for t in range(NB):
                    off = 2 + NB * j + t
                    finish_chunk(off, (2 + t) % NB, off + 2, True, True)
                return 0

            lax.fori_loop(0, m4, body, 0)
            for t in range(rem):
                off = 2 + NB * m4 + t
                finish_chunk(off, (2 + t) % NB, off + 2, off + 2 <= n - 1, True)
            drain_scatter((n - 2) % NB)
            drain_scatter((n - 1) % NB)

        for base, n in SEGMENTS:
            run_segment(base, n)
        pl.when(wid == NW - 1)(lambda: run_segment(*SEG_EXTRA))
        plsc.subcore_barrier()

        pltpu.sync_copy(hist_v, hist_out.at[wid])

        def write_rows(nrows):
            pltpu.sync_copy(feat_acc.at[pl.ds(row0, nrows)],
                            feat_out.at[cid].at[pl.ds(row0, nrows)])

        pl.when(sid < NS - 1)(lambda: write_rows(ROWS_A))
        pl.when(sid == NS - 1)(lambda: write_rows(ROWS_LAST))

    return sc_kernel(x, ei)


BR = 1024


def _tc_body(fp_ref, h_ref, p_ref, m_ref, x_ref, wl_ref, bl_ref, wr_ref, o_ref):
    feat = fp_ref[0] + fp_ref[1]
    dn0 = (((1,), (0,)), ((), ()))
    cnt8 = jnp.sum(h_ref[...], axis=0)
    tmp = lax.dot_general(p_ref[...], cnt8, dn0, preferred_element_type=jnp.float32)
    cnt = jnp.sum(tmp * m_ref[...], axis=1, keepdims=True)
    mean = feat / jnp.maximum(cnt, 1.0)
    dn = (((1,), (1,)), ((), ()))
    o_ref[...] = (
        lax.dot_general(mean, wl_ref[...], dn, preferred_element_type=jnp.float32)
        + lax.dot_general(x_ref[...], wr_ref[...], dn, preferred_element_type=jnp.float32)
        + bl_ref[...]
    )


def _tc_finish(feat_p, hist_p, P, M, x, W_l, b_l2, W_r):
    grid = (pl.cdiv(N_NODES, BR),)
    return pl.pallas_call(
        _tc_body,
        grid=grid,
        in_specs=[
            pl.BlockSpec((NC, BR, D), lambda i: (0, i, 0)),
            pl.BlockSpec((NW, BR // D, D), lambda i: (0, i, 0)),
            pl.BlockSpec((BR, BR // D), lambda i: (0, 0)),
            pl.BlockSpec((BR, D), lambda i: (0, 0)),
            pl.BlockSpec((BR, D), lambda i: (i, 0)),
            pl.BlockSpec((D, D), lambda i: (0, 0)),
            pl.BlockSpec((1, D), lambda i: (0, 0)),
            pl.BlockSpec((D, D), lambda i: (0, 0)),
        ],
        out_specs=pl.BlockSpec((BR, D), lambda i: (i, 0)),
        out_shape=jax.ShapeDtypeStruct((N_NODES, D), jnp.float32),
    )(feat_p, hist_p, P, M, x, W_l, b_l2, W_r)


@jax.jit
def kernel(x, edge_index, W_l, b_l, W_r):
    feat_p, hist_p = _sc_accumulate(x, edge_index.astype(jnp.int32))
    ar = jnp.arange(BR, dtype=jnp.int32)[:, None]
    P = (ar // D == jnp.arange(BR // D, dtype=jnp.int32)[None, :]).astype(jnp.float32)
    M = (ar % D == jnp.arange(D, dtype=jnp.int32)[None, :]).astype(jnp.float32)
    return _tc_finish(feat_p, hist_p, P, M, x, W_l, b_l.reshape(1, D), W_r)


# speedup vs baseline: 1.1247x; 1.1247x over previous
import functools

import jax
import jax.numpy as jnp
from jax import lax
from jax.experimental import pallas as pl
from jax.experimental.pallas import tpu as pltpu
from jax.experimental.pallas import tpu_sc as plsc

N_NODES = 10000
D = 128
N_EDGES = 320000

NC = 2
NS = 16
NW = NC * NS
EDGES_PER_TILE = N_EDGES // NW
CHUNK = 80
N_CHUNKS = EDGES_PER_TILE // CHUNK
SEGMENTS = ((0, 60), (60, 60), (120, 5))
SEG_MAX = 60
ROWS_A = 632
ROWS_LAST = N_NODES - (NS - 1) * ROWS_A
ZROWS = 8
HR = 80


def _sc_accumulate(x, ei):
    mesh = plsc.VectorSubcoreMesh(core_axis_name="c", subcore_axis_name="s")

    @functools.partial(
        pl.kernel,
        out_type=(
            jax.ShapeDtypeStruct((NC, N_NODES, D), jnp.float32),
            jax.ShapeDtypeStruct((NW, HR, D), jnp.float32),
        ),
        mesh=mesh,
        compiler_params=pltpu.CompilerParams(needs_layout_passes=False),
        scratch_types=[
            pltpu.VMEM_SHARED((N_NODES, D), jnp.float32),
            pltpu.VMEM((SEG_MAX * CHUNK,), jnp.int32),
            pltpu.VMEM((SEG_MAX * CHUNK,), jnp.int32),
            pltpu.VMEM((CHUNK, D), jnp.float32),
            pltpu.VMEM((CHUNK, D), jnp.float32),
            pltpu.VMEM((CHUNK, D), jnp.float32),
            pltpu.VMEM((HR, D), jnp.float32),
            pltpu.SemaphoreType.DMA,
            pltpu.SemaphoreType.DMA,
            pltpu.SemaphoreType.DMA,
            pltpu.SemaphoreType.DMA,
            pltpu.SemaphoreType.DMA,
            pltpu.SemaphoreType.DMA,
        ],
    )
    def sc_kernel(x_hbm, ei_hbm,
                  feat_out, hist_out,
                  feat_acc, src_all, dst_all,
                  rows_v0, rows_v1, rows_v2, hist_v,
                  sem0, sem1, sem2, ssem0, ssem1, ssem2):
        cid = lax.axis_index("c")
        sid = lax.axis_index("s")
        wid = cid * NS + sid

        zvec = jnp.zeros((16,), jnp.float32)
        for i in range(ZROWS):
            for j in range(D // 16):
                rows_v0[i, pl.ds(j * 16, 16)] = zvec

        def zero_hist(i, _):
            for j in range(D // 16):
                hist_v[i, pl.ds(j * 16, 16)] = zvec
            return 0
        lax.fori_loop(0, HR, zero_hist, 0)

        row0 = sid * ROWS_A

        def zero_rows(nrows):
            def zbody(k, _):
                pltpu.sync_copy(rows_v0.at[pl.ds(0, ZROWS)],
                                feat_acc.at[pl.ds(row0 + k * ZROWS, ZROWS)])
                return 0
            lax.fori_loop(0, nrows // ZROWS, zbody, 0)

        pl.when(sid < NS - 1)(lambda: zero_rows(ROWS_A))
        pl.when(sid == NS - 1)(lambda: zero_rows(ROWS_LAST))
        plsc.subcore_barrier()

        edge_base = wid * EDGES_PER_TILE
        ones16 = jnp.ones((16,), jnp.float32)
        bufs = ((rows_v0, sem0, ssem0), (rows_v1, sem1, ssem1),
                (rows_v2, sem2, ssem2))

        def start_gather(off, b):
            rows_v, sem, _ = bufs[b]
            pltpu.async_copy(
                x_hbm.at[src_all.at[pl.ds(off * CHUNK, CHUNK)]],
                rows_v, sem)

        def drain_scatter(b):
            rows_v, _, ssem = bufs[b]
            for k in range(CHUNK // 16):
                pltpu.make_async_copy(x_hbm.at[pl.ds(0, 16)],
                                      rows_v.at[pl.ds(k * 16, 16)], ssem).wait()

        def finish_chunk(off, b, nxt_off, start_next, drain_prev):
            rows_v, sem, ssem = bufs[b]
            pltpu.make_async_copy(x_hbm.at[pl.ds(0, CHUNK)], rows_v, sem).wait()
            for k in range(CHUNK // 16):
                dv = dst_all[pl.ds(off * CHUNK + k * 16, 16)]
                pltpu.async_copy(rows_v.at[pl.ds(k * 16, 16)],
                                 feat_acc.at[dv], ssem, add=True)
                hi = lax.shift_right_logical(dv, 7)
                lo = lax.bitwise_and(dv, 127)
                plsc.addupdate_scatter(hist_v, [hi, lo], ones16)
            if drain_prev:
                drain_scatter((b + 2) % 3)
            if start_next:
                start_gather(nxt_off, (b + 2) % 3)

        for base, n in SEGMENTS:
            pltpu.sync_copy(
                ei_hbm.at[pl.ds(edge_base + base * CHUNK, n * CHUNK)],
                src_all.at[pl.ds(0, n * CHUNK)])
            pltpu.sync_copy(
                ei_hbm.at[pl.ds(N_EDGES + edge_base + base * CHUNK, n * CHUNK)],
                dst_all.at[pl.ds(0, n * CHUNK)])
            start_gather(0, 0)
            start_gather(1, 1)
            finish_chunk(0, 0, 2, n > 2, False)
            finish_chunk(1, 1, 3, n > 3, True)
            m3 = (n - 4) // 3 if n > 4 else 0
            rem = (n - 2) - 3 * m3

            def body(j, _):
                for t in range(3):
                    off = 2 + 3 * j + t
                    finish_chunk(off, (2 + t) % 3, off + 2, True, True)
                return 0

            lax.fori_loop(0, m3, body, 0)
            for t in range(rem):
                off = 2 + 3 * m3 + t
                finish_chunk(off, (2 + t) % 3, off + 2, off + 2 <= n - 1, True)
            drain_scatter((n - 1) % 3)
        plsc.subcore_barrier()

        pltpu.sync_copy(hist_v, hist_out.at[wid])

        def write_rows(nrows):
            pltpu.sync_copy(feat_acc.at[pl.ds(row0, nrows)],
                            feat_out.at[cid].at[pl.ds(row0, nrows)])

        pl.when(sid < NS - 1)(lambda: write_rows(ROWS_A))
        pl.when(sid == NS - 1)(lambda: write_rows(ROWS_LAST))

    return sc_kernel(x, ei)


BR = 1024


def _tc_body(fp_ref, h_ref, p_ref, m_ref, x_ref, wl_ref, bl_ref, wr_ref, o_ref):
    feat = fp_ref[0] + fp_ref[1]
    dn0 = (((1,), (0,)), ((), ()))
    cnt8 = jnp.sum(h_ref[...], axis=0)
    tmp = lax.dot_general(p_ref[...], cnt8, dn0, preferred_element_type=jnp.float32)
    cnt = jnp.sum(tmp * m_ref[...], axis=1, keepdims=True)
    mean = feat / jnp.maximum(cnt, 1.0)
    dn = (((1,), (1,)), ((), ()))
    o_ref[...] = (
        lax.dot_general(mean, wl_ref[...], dn, preferred_element_type=jnp.float32)
        + lax.dot_general(x_ref[...], wr_ref[...], dn, preferred_element_type=jnp.float32)
        + bl_ref[...]
    )


def _tc_finish(feat_p, hist_p, P, M, x, W_l, b_l2, W_r):
    grid = (pl.cdiv(N_NODES, BR),)
    return pl.pallas_call(
        _tc_body,
        grid=grid,
        in_specs=[
            pl.BlockSpec((NC, BR, D), lambda i: (0, i, 0)),
            pl.BlockSpec((NW, BR // D, D), lambda i: (0, i, 0)),
            pl.BlockSpec((BR, BR // D), lambda i: (0, 0)),
            pl.BlockSpec((BR, D), lambda i: (0, 0)),
            pl.BlockSpec((BR, D), lambda i: (i, 0)),
            pl.BlockSpec((D, D), lambda i: (0, 0)),
            pl.BlockSpec((1, D), lambda i: (0, 0)),
            pl.BlockSpec((D, D), lambda i: (0, 0)),
        ],
        out_specs=pl.BlockSpec((BR, D), lambda i: (i, 0)),
        out_shape=jax.ShapeDtypeStruct((N_NODES, D), jnp.float32),
    )(feat_p, hist_p, P, M, x, W_l, b_l2, W_r)


@jax.jit
def kernel(x, edge_index, W_l, b_l, W_r):
    ei_flat = edge_index.astype(jnp.int32).reshape(2 * N_EDGES)
    feat_p, hist_p = _sc_accumulate(x, ei_flat)
    ar = jnp.arange(BR, dtype=jnp.int32)[:, None]
    P = (ar // D == jnp.arange(BR // D, dtype=jnp.int32)[None, :]).astype(jnp.float32)
    M = (ar % D == jnp.arange(D, dtype=jnp.int32)[None, :]).astype(jnp.float32)
    return _tc_finish(feat_p, hist_p, P, M, x, W_l, b_l.reshape(1, D), W_r)


# speedup vs baseline: 1.1579x; 1.0296x over previous
import functools

import jax
import jax.numpy as jnp
from jax import lax
from jax.experimental import pallas as pl
from jax.experimental.pallas import tpu as pltpu
from jax.experimental.pallas import tpu_sc as plsc

N_NODES = 10000
D = 128
N_EDGES = 320000

NC = 2
NS = 16
NW = NC * NS
EDGES_PER_TILE = N_EDGES // NW
CHUNK = 80
N_CHUNKS = EDGES_PER_TILE // CHUNK
SEGMENTS = ((0, 60), (60, 60), (120, 5))
SEG_MAX = 60
ROWS_A = 632
ROWS_LAST = N_NODES - (NS - 1) * ROWS_A
ZROWS = 8
HR = 80


def _sc_accumulate(x, ei):
    mesh = plsc.VectorSubcoreMesh(core_axis_name="c", subcore_axis_name="s")

    @functools.partial(
        pl.kernel,
        out_type=(
            jax.ShapeDtypeStruct((NC, N_NODES, D), jnp.float32),
            jax.ShapeDtypeStruct((NW, HR, D), jnp.float32),
        ),
        mesh=mesh,
        compiler_params=pltpu.CompilerParams(needs_layout_passes=False),
        scratch_types=[
            pltpu.VMEM_SHARED((N_NODES, D), jnp.float32),
            pltpu.VMEM((SEG_MAX * CHUNK,), jnp.int32),
            pltpu.VMEM((SEG_MAX * CHUNK,), jnp.int32),
            pltpu.VMEM((CHUNK, D), jnp.float32),
            pltpu.VMEM((CHUNK, D), jnp.float32),
            pltpu.VMEM((CHUNK, D), jnp.float32),
            pltpu.VMEM((HR, D), jnp.float32),
            pltpu.SemaphoreType.DMA,
            pltpu.SemaphoreType.DMA,
            pltpu.SemaphoreType.DMA,
            pltpu.SemaphoreType.DMA,
            pltpu.SemaphoreType.DMA,
            pltpu.SemaphoreType.DMA,
        ],
    )
    def sc_kernel(x_hbm, ei_hbm,
                  feat_out, hist_out,
                  feat_acc, src_all, dst_all,
                  rows_v0, rows_v1, rows_v2, hist_v,
                  sem0, sem1, sem2, ssem0, ssem1, ssem2):
        cid = lax.axis_index("c")
        sid = lax.axis_index("s")
        wid = cid * NS + sid

        zvec = jnp.zeros((16,), jnp.float32)
        def zrow(i, _):
            for j in range(D // 16):
                rows_v0[i, pl.ds(j * 16, 16)] = zvec
            return 0
        lax.fori_loop(0, CHUNK, zrow, 0)

        def zero_hist(i, _):
            for j in range(D // 16):
                hist_v[i, pl.ds(j * 16, 16)] = zvec
            return 0
        lax.fori_loop(0, HR, zero_hist, 0)

        row0 = sid * ROWS_A

        def zero_rows(nrows):
            full = nrows // CHUNK
            pend = []
            for k in range(full):
                pend.append(pltpu.async_copy(
                    rows_v0, feat_acc.at[pl.ds(row0 + k * CHUNK, CHUNK)],
                    sem0))
            r = nrows - full * CHUNK
            if r:
                pend.append(pltpu.async_copy(
                    rows_v0.at[pl.ds(0, r)],
                    feat_acc.at[pl.ds(row0 + full * CHUNK, r)], sem0))
            for d in pend:
                d.wait()

        pl.when(sid < NS - 1)(lambda: zero_rows(ROWS_A))
        pl.when(sid == NS - 1)(lambda: zero_rows(ROWS_LAST))
        plsc.subcore_barrier()

        edge_base = wid * EDGES_PER_TILE
        ones16 = jnp.ones((16,), jnp.float32)
        bufs = ((rows_v0, sem0, ssem0), (rows_v1, sem1, ssem1),
                (rows_v2, sem2, ssem2))

        def start_gather(off, b):
            rows_v, sem, _ = bufs[b]
            pltpu.async_copy(
                x_hbm.at[src_all.at[pl.ds(off * CHUNK, CHUNK)]],
                rows_v, sem)

        def drain_scatter(b):
            rows_v, _, ssem = bufs[b]
            for k in range(CHUNK // 16):
                pltpu.make_async_copy(x_hbm.at[pl.ds(0, 16)],
                                      rows_v.at[pl.ds(k * 16, 16)], ssem).wait()

        def finish_chunk(off, b, nxt_off, start_next, drain_prev):
            rows_v, sem, ssem = bufs[b]
            pltpu.make_async_copy(x_hbm.at[pl.ds(0, CHUNK)], rows_v, sem).wait()
            for k in range(CHUNK // 16):
                dv = dst_all[pl.ds(off * CHUNK + k * 16, 16)]
                pltpu.async_copy(rows_v.at[pl.ds(k * 16, 16)],
                                 feat_acc.at[dv], ssem, add=True)
                hi = lax.shift_right_logical(dv, 7)
                lo = lax.bitwise_and(dv, 127)
                plsc.addupdate_scatter(hist_v, [hi, lo], ones16)
            if drain_prev:
                drain_scatter((b + 2) % 3)
            if start_next:
                start_gather(nxt_off, (b + 2) % 3)

        for base, n in SEGMENTS:
            pltpu.sync_copy(
                ei_hbm.at[pl.ds(edge_base + base * CHUNK, n * CHUNK)],
                src_all.at[pl.ds(0, n * CHUNK)])
            pltpu.sync_copy(
                ei_hbm.at[pl.ds(N_EDGES + edge_base + base * CHUNK, n * CHUNK)],
                dst_all.at[pl.ds(0, n * CHUNK)])
            start_gather(0, 0)
            start_gather(1, 1)
            finish_chunk(0, 0, 2, n > 2, False)
            finish_chunk(1, 1, 3, n > 3, True)
            m3 = (n - 4) // 3 if n > 4 else 0
            rem = (n - 2) - 3 * m3

            def body(j, _):
                for t in range(3):
                    off = 2 + 3 * j + t
                    finish_chunk(off, (2 + t) % 3, off + 2, True, True)
                return 0

            lax.fori_loop(0, m3, body, 0)
            for t in range(rem):
                off = 2 + 3 * m3 + t
                finish_chunk(off, (2 + t) % 3, off + 2, off + 2 <= n - 1, True)
            drain_scatter((n - 1) % 3)
        plsc.subcore_barrier()

        pltpu.sync_copy(hist_v, hist_out.at[wid])

        def write_rows(nrows):
            pltpu.sync_copy(feat_acc.at[pl.ds(row0, nrows)],
                            feat_out.at[cid].at[pl.ds(row0, nrows)])

        pl.when(sid < NS - 1)(lambda: write_rows(ROWS_A))
        pl.when(sid == NS - 1)(lambda: write_rows(ROWS_LAST))

    return sc_kernel(x, ei)


BR = 1024


def _tc_body(fp_ref, h_ref, x_ref, wl_ref, bl_ref, wr_ref, o_ref):
    feat = fp_ref[0] + fp_ref[1]
    ri = lax.broadcasted_iota(jnp.int32, (BR, BR // D), 0)
    pm = (lax.shift_right_logical(ri, 7) ==
          lax.broadcasted_iota(jnp.int32, (BR, BR // D), 1)).astype(jnp.float32)
    rj = lax.broadcasted_iota(jnp.int32, (BR, D), 0)
    mm = (lax.bitwise_and(rj, D - 1) ==
          lax.broadcasted_iota(jnp.int32, (BR, D), 1)).astype(jnp.float32)
    dn0 = (((1,), (0,)), ((), ()))
    cnt8 = jnp.sum(h_ref[...], axis=0)
    tmp = lax.dot_general(pm, cnt8, dn0, preferred_element_type=jnp.float32)
    cnt = jnp.sum(tmp * mm, axis=1, keepdims=True)
    mean = feat / jnp.maximum(cnt, 1.0)
    dn = (((1,), (1,)), ((), ()))
    o_ref[...] = (
        lax.dot_general(mean, wl_ref[...], dn, preferred_element_type=jnp.float32)
        + lax.dot_general(x_ref[...], wr_ref[...], dn, preferred_element_type=jnp.float32)
        + bl_ref[...]
    )


def _tc_finish(feat_p, hist_p, x, W_l, b_l2, W_r):
    grid = (pl.cdiv(N_NODES, BR),)
    return pl.pallas_call(
        _tc_body,
        grid=grid,
        in_specs=[
            pl.BlockSpec((NC, BR, D), lambda i: (0, i, 0)),
            pl.BlockSpec((NW, BR // D, D), lambda i: (0, i, 0)),
            pl.BlockSpec((BR, D), lambda i: (i, 0)),
            pl.BlockSpec((D, D), lambda i: (0, 0)),
            pl.BlockSpec((1, D), lambda i: (0, 0)),
            pl.BlockSpec((D, D), lambda i: (0, 0)),
        ],
        out_specs=pl.BlockSpec((BR, D), lambda i: (i, 0)),
        out_shape=jax.ShapeDtypeStruct((N_NODES, D), jnp.float32),
    )(feat_p, hist_p, x, W_l, b_l2, W_r)


@jax.jit
def kernel(x, edge_index, W_l, b_l, W_r):
    ei_flat = edge_index.astype(jnp.int32).reshape(2 * N_EDGES)
    feat_p, hist_p = _sc_accumulate(x, ei_flat)
    return _tc_finish(feat_p, hist_p, x, W_l, b_l.reshape(1, D), W_r)


# speedup vs baseline: 1.1837x; 1.0222x over previous
import functools

import jax
import jax.numpy as jnp
from jax import lax
from jax.experimental import pallas as pl
from jax.experimental.pallas import tpu as pltpu
from jax.experimental.pallas import tpu_sc as plsc

N_NODES = 10000
D = 128
N_EDGES = 320000

NC = 2
NS = 16
NW = NC * NS
EDGES_PER_TILE = N_EDGES // NW
CHUNK = 80
N_CHUNKS = EDGES_PER_TILE // CHUNK
SEGMENTS = ((0, 60), (60, 60), (120, 5))
SEG_MAX = 60
ROWS_A = 632
ROWS_LAST = N_NODES - (NS - 1) * ROWS_A
ZROWS = 8
HR = 80


def _sc_accumulate(x, ei):
    mesh = plsc.VectorSubcoreMesh(core_axis_name="c", subcore_axis_name="s")

    @functools.partial(
        pl.kernel,
        out_type=(
            jax.ShapeDtypeStruct((NC, N_NODES, D), jnp.float32),
            jax.ShapeDtypeStruct((NW, HR, D), jnp.float32),
        ),
        mesh=mesh,
        compiler_params=pltpu.CompilerParams(needs_layout_passes=False),
        scratch_types=[
            pltpu.VMEM_SHARED((N_NODES, D), jnp.float32),
            pltpu.VMEM((SEG_MAX * CHUNK,), jnp.int32),
            pltpu.VMEM((SEG_MAX * CHUNK,), jnp.int32),
            pltpu.VMEM((CHUNK, D), jnp.float32),
            pltpu.VMEM((CHUNK, D), jnp.float32),
            pltpu.VMEM((CHUNK, D), jnp.float32),
            pltpu.VMEM((HR, D), jnp.float32),
            pltpu.SemaphoreType.DMA,
            pltpu.SemaphoreType.DMA,
            pltpu.SemaphoreType.DMA,
            pltpu.SemaphoreType.DMA,
            pltpu.SemaphoreType.DMA,
            pltpu.SemaphoreType.DMA,
        ],
    )
    def sc_kernel(x_hbm, ei_hbm,
                  feat_out, hist_out,
                  feat_acc, src_all, dst_all,
                  rows_v0, rows_v1, rows_v2, hist_v,
                  sem0, sem1, sem2, ssem0, ssem1, ssem2):
        cid = lax.axis_index("c")
        sid = lax.axis_index("s")
        wid = cid * NS + sid

        zvec = jnp.zeros((16,), jnp.float32)
        def zrow(i, _):
            for j in range(D // 16):
                rows_v0[i, pl.ds(j * 16, 16)] = zvec
            return 0
        lax.fori_loop(0, CHUNK, zrow, 0)

        def zero_hist(i, _):
            for j in range(D // 16):
                hist_v[i, pl.ds(j * 16, 16)] = zvec
            return 0
        lax.fori_loop(0, HR, zero_hist, 0)

        row0 = sid * ROWS_A

        def zero_rows(nrows):
            full = nrows // CHUNK
            pend = []
            for k in range(full):
                pend.append(pltpu.async_copy(
                    rows_v0, feat_acc.at[pl.ds(row0 + k * CHUNK, CHUNK)],
                    sem0))
            r = nrows - full * CHUNK
            if r:
                pend.append(pltpu.async_copy(
                    rows_v0.at[pl.ds(0, r)],
                    feat_acc.at[pl.ds(row0 + full * CHUNK, r)], sem0))
            for d in pend:
                d.wait()

        pl.when(sid < NS - 1)(lambda: zero_rows(ROWS_A))
        pl.when(sid == NS - 1)(lambda: zero_rows(ROWS_LAST))
        plsc.subcore_barrier()

        edge_base = wid * EDGES_PER_TILE
        ones16 = jnp.ones((16,), jnp.float32)
        bufs = ((rows_v0, sem0, ssem0), (rows_v1, sem1, ssem1),
                (rows_v2, sem2, ssem2))

        def start_gather(off, b):
            rows_v, sem, _ = bufs[b]
            pltpu.async_copy(
                x_hbm.at[src_all.at[pl.ds(off * CHUNK, CHUNK)]],
                rows_v, sem)

        def drain_scatter(b):
            rows_v, _, ssem = bufs[b]
            for k in range(CHUNK // 16):
                pltpu.make_async_copy(x_hbm.at[pl.ds(0, 16)],
                                      rows_v.at[pl.ds(k * 16, 16)], ssem).wait()

        def finish_chunk(off, b, nxt_off, start_next, drain_prev):
            rows_v, sem, ssem = bufs[b]
            pltpu.make_async_copy(x_hbm.at[pl.ds(0, CHUNK)], rows_v, sem).wait()
            for k in range(CHUNK // 16):
                dv = dst_all[pl.ds(off * CHUNK + k * 16, 16)]
                pltpu.async_copy(rows_v.at[pl.ds(k * 16, 16)],
                                 feat_acc.at[dv], ssem, add=True)
                hi = lax.shift_right_logical(dv, 7)
                lo = lax.bitwise_and(dv, 127)
                plsc.addupdate_scatter(hist_v, [hi, lo], ones16)
            if drain_prev:
                drain_scatter((b + 2) % 3)
            if start_next:
                start_gather(nxt_off, (b + 2) % 3)

        for base, n in SEGMENTS:
            pltpu.sync_copy(
                ei_hbm.at[pl.ds(edge_base + base * CHUNK, n * CHUNK)],
                src_all.at[pl.ds(0, n * CHUNK)])
            pltpu.sync_copy(
                ei_hbm.at[pl.ds(N_EDGES + edge_base + base * CHUNK, n * CHUNK)],
                dst_all.at[pl.ds(0, n * CHUNK)])
            start_gather(0, 0)
            start_gather(1, 1)
            finish_chunk(0, 0, 2, n > 2, False)
            finish_chunk(1, 1, 3, n > 3, True)
            m3 = (n - 4) // 3 if n > 4 else 0
            rem = (n - 2) - 3 * m3

            def body(j, _):
                for t in range(3):
                    off = 2 + 3 * j + t
                    finish_chunk(off, (2 + t) % 3, off + 2, True, True)
                return 0

            lax.fori_loop(0, m3, body, 0)
            for t in range(rem):
                off = 2 + 3 * m3 + t
                finish_chunk(off, (2 + t) % 3, off + 2, off + 2 <= n - 1, True)
            drain_scatter((n - 1) % 3)
        plsc.subcore_barrier()

        pltpu.sync_copy(hist_v, hist_out.at[wid])

        def write_rows(nrows):
            pltpu.sync_copy(feat_acc.at[pl.ds(row0, nrows)],
                            feat_out.at[cid].at[pl.ds(row0, nrows)])

        pl.when(sid < NS - 1)(lambda: write_rows(ROWS_A))
        pl.when(sid == NS - 1)(lambda: write_rows(ROWS_LAST))

    return sc_kernel(x, ei)


BR = 2048


def _tc_body(fp_ref, h_ref, x_ref, wl_ref, bl_ref, wr_ref, o_ref):
    feat = fp_ref[0] + fp_ref[1]
    ri = lax.broadcasted_iota(jnp.int32, (BR, BR // D), 0)
    pm = (lax.shift_right_logical(ri, 7) ==
          lax.broadcasted_iota(jnp.int32, (BR, BR // D), 1)).astype(jnp.float32)
    rj = lax.broadcasted_iota(jnp.int32, (BR, D), 0)
    mm = (lax.bitwise_and(rj, D - 1) ==
          lax.broadcasted_iota(jnp.int32, (BR, D), 1)).astype(jnp.float32)
    dn0 = (((1,), (0,)), ((), ()))
    cnt8 = jnp.sum(h_ref[...], axis=0)
    tmp = lax.dot_general(pm, cnt8, dn0, preferred_element_type=jnp.float32)
    cnt = jnp.sum(tmp * mm, axis=1, keepdims=True)
    mean = feat / jnp.maximum(cnt, 1.0)
    dn = (((1,), (1,)), ((), ()))
    o_ref[...] = (
        lax.dot_general(mean, wl_ref[...], dn, preferred_element_type=jnp.float32)
        + lax.dot_general(x_ref[...], wr_ref[...], dn, preferred_element_type=jnp.float32)
        + bl_ref[...]
    )


def _tc_finish(feat_p, hist_p, x, W_l, b_l2, W_r):
    grid = (pl.cdiv(N_NODES, BR),)
    return pl.pallas_call(
        _tc_body,
        grid=grid,
        in_specs=[
            pl.BlockSpec((NC, BR, D), lambda i: (0, i, 0)),
            pl.BlockSpec((NW, BR // D, D), lambda i: (0, i, 0)),
            pl.BlockSpec((BR, D), lambda i: (i, 0)),
            pl.BlockSpec((D, D), lambda i: (0, 0)),
            pl.BlockSpec((1, D), lambda i: (0, 0)),
            pl.BlockSpec((D, D), lambda i: (0, 0)),
        ],
        out_specs=pl.BlockSpec((BR, D), lambda i: (i, 0)),
        out_shape=jax.ShapeDtypeStruct((N_NODES, D), jnp.float32),
    )(feat_p, hist_p, x, W_l, b_l2, W_r)


@jax.jit
def kernel(x, edge_index, W_l, b_l, W_r):
    ei_flat = edge_index.astype(jnp.int32).reshape(2 * N_EDGES)
    feat_p, hist_p = _sc_accumulate(x, ei_flat)
    return _tc_finish(feat_p, hist_p, x, W_l, b_l.reshape(1, D), W_r)
